# baseline (speedup 1.0000x reference)
import functools

import jax
import jax.numpy as jnp
import numpy as np
from jax import lax
from jax.experimental import pallas as pl
from jax.experimental.pallas import tpu as pltpu
from jax.experimental.pallas import tpu_sc as plsc

HID = 128
TIME_DIM = 128

BLK_E = 512
BLK_N = 1000

NC, NS = 2, 16
CHK = 200
CHKS = 80


def _silu(x):
    return x * jax.nn.sigmoid(x)


def _dot(a, b):
    return jnp.dot(a, b, preferred_element_type=jnp.float32)


def _wspec(shape):
    return pl.BlockSpec(shape, lambda i: (0,) * len(shape))


def _rspec(blk, ncols):
    return pl.BlockSpec((blk, ncols), lambda i: (i, 0))


def _node_enc_body(x_ref, wne, bne, h_ref, hbf_ref):
    h = _dot(x_ref[...], wne[...]) + bne[...]
    h_ref[...] = h
    hbf_ref[...] = h.astype(jnp.bfloat16)


def _edge_enc_body(ea_ref, wee, bee, e_ref):
    e_ref[...] = _dot(ea_ref[...], wee[...]) + bee[...]


def _edge_body(hs_ref, hd_ref, e_ref, rel_ref, w1hh, w1de, b1, w2, b2, wc, bc,
               m_ref, eo_ref, pr_ref):
    e = e_ref[...]
    rel = rel_ref[...]
    d2 = jnp.sum(rel * rel, axis=1, keepdims=True)
    hh = jnp.concatenate([hs_ref[...].astype(jnp.float32),
                          hd_ref[...].astype(jnp.float32)], axis=1)
    de = jnp.concatenate(
        [d2, e, jnp.zeros((e.shape[0], HID - 1), jnp.float32)], axis=1)
    t1 = _dot(hh, w1hh[...]) + _dot(de, w1de[...]) + b1[...]
    u = _silu(t1)
    m = _silu(_dot(u, w2[...]) + b2[...])
    cw = _dot(m, wc[...]) + bc[...]
    m_ref[...] = m
    eo_ref[...] = e + m
    one4 = (lax.broadcasted_iota(jnp.int32, (1, 16), 1) == 4).astype(jnp.float32)
    pr_ref[...] = rel * cw + one4


def _node_body(h_ref, a0_ref, a1_ref, pos_ref, p0_ref, p1_ref,
               wn1, bn1, wn2, bn2, h_o, pos_o, hbf_o):
    h = h_ref[...]
    agg = a0_ref[...] + a1_ref[...]
    pu = p0_ref[...] + p1_ref[...]
    hcat = jnp.concatenate([h, agg], axis=1)
    u = _silu(_dot(hcat, wn1[...]) + bn1[...])
    hn = h + _dot(u, wn2[...]) + bn2[...]
    h_o[...] = hn
    hbf_o[...] = hn.astype(jnp.bfloat16)
    cnt = jnp.maximum(pu[:, 4:5], 1.0)
    mask3 = (lax.broadcasted_iota(jnp.int32, (1, 16), 1) < 3).astype(jnp.float32)
    pos_o[...] = pos_ref[...] + pu * mask3 / cnt


def _final_node_body(h_ref, te_ref,
                     wat1, bat1, wat2, bat2, wat3, bat3,
                     wp1, bp1, wp2, bp2, wp3, bp3,
                     wbs, wbte, bb1, wbt,
                     atom_o, pos_o, s_o, t_o):
    h = h_ref[...]
    te = te_ref[...]
    hcat = jnp.concatenate([h, te], axis=1)
    a1 = _silu(_dot(hcat, wat1[...]) + bat1[...])
    a2 = _silu(_dot(a1, wat2[...]) + bat2[...])
    atom_o[...] = _dot(a2, wat3[...]) + bat3[...]
    p1 = _silu(_dot(hcat, wp1[...]) + bp1[...])
    p2 = _silu(_dot(p1, wp2[...]) + bp2[...])
    pos_o[...] = _dot(p2, wp3[...]) + bp3[...]
    s_o[...] = _dot(h, wbs[...]) + _dot(te, wbte[...]) + bb1[...]
    t_o[...] = _dot(h, wbt[...])


def _bond_body(zb_ref, w2, b2, w3, b3, o_ref):
    u = _silu(zb_ref[...])
    u2 = _silu(_dot(u, w2[...]) + b2[...])
    o_ref[...] = _dot(u2, w3[...]) + b3[...]


def _f32(shape):
    return jax.ShapeDtypeStruct(shape, jnp.float32)


def _node_enc(x, wne, bne):
    n = x.shape[0]
    return pl.pallas_call(
        _node_enc_body,
        grid=(n // BLK_N,),
        in_specs=[_rspec(BLK_N, HID), _wspec((HID, HID)), _wspec((1, HID))],
        out_specs=[_rspec(BLK_N, HID), _rspec(BLK_N, HID)],
        out_shape=[_f32((n, HID)), jax.ShapeDtypeStruct((n, HID), jnp.bfloat16)],
    )(x, wne, bne)


def _edge_enc(ea, wee, bee):
    ne = ea.shape[0]
    return pl.pallas_call(
        _edge_enc_body,
        grid=(ne // BLK_E,),
        in_specs=[_rspec(BLK_E, 16), _wspec((16, HID)), _wspec((1, HID))],
        out_specs=_rspec(BLK_E, HID),
        out_shape=_f32((ne, HID)),
    )(ea, wee, bee)


def _edge_pass(hs, hd, e, rel16, w1hh, w1de, b1, w2, b2, wc, bc):
    ne = e.shape[0]
    return pl.pallas_call(
        _edge_body,
        grid=(ne // BLK_E,),
        in_specs=[_rspec(BLK_E, HID), _rspec(BLK_E, HID), _rspec(BLK_E, HID),
                  _rspec(BLK_E, 16),
                  _wspec((2 * HID, HID)), _wspec((2 * HID, HID)), _wspec((1, HID)),
                  _wspec((HID, HID)), _wspec((1, HID)),
                  _wspec((HID, 1)), _wspec((1, 1))],
        out_specs=[_rspec(BLK_E, HID), _rspec(BLK_E, HID), _rspec(BLK_E, 16)],
        out_shape=[_f32((ne, HID)), _f32((ne, HID)), _f32((ne, 16))],
    )(hs, hd, e, rel16, w1hh, w1de, b1, w2, b2, wc, bc)


def _node_pass(h, agg2, pos4, pu2, wn1, bn1, wn2, bn2):
    n = h.shape[0]
    return pl.pallas_call(
        _node_body,
        grid=(n // BLK_N,),
        in_specs=[_rspec(BLK_N, HID), _rspec(BLK_N, HID), _rspec(BLK_N, HID),
                  _rspec(BLK_N, 16), _rspec(BLK_N, 16), _rspec(BLK_N, 16),
                  _wspec((2 * HID, HID)), _wspec((1, HID)),
                  _wspec((HID, HID)), _wspec((1, HID))],
        out_specs=[_rspec(BLK_N, HID), _rspec(BLK_N, 16), _rspec(BLK_N, HID)],
        out_shape=[_f32((n, HID)), _f32((n, 16)),
                   jax.ShapeDtypeStruct((n, HID), jnp.bfloat16)],
    )(h, agg2[0], agg2[1], pos4, pu2[0], pu2[1], wn1, bn1, wn2, bn2)


def _final_node(h, te, *ws):
    n = h.shape[0]
    wspecs = [_wspec(w.shape) for w in ws]
    return pl.pallas_call(
        _final_node_body,
        grid=(n // BLK_N,),
        in_specs=[_rspec(BLK_N, HID), _rspec(BLK_N, HID)] + wspecs,
        out_specs=[_rspec(BLK_N, HID), _rspec(BLK_N, 8),
                   _rspec(BLK_N, HID), _rspec(BLK_N, HID)],
        out_shape=[_f32((n, HID)), _f32((n, 8)), _f32((n, HID)), _f32((n, HID))],
    )(h, te, *ws)


def _bond_pass(zb, w2, b2, w3, b3):
    ne = zb.shape[0]
    return pl.pallas_call(
        _bond_body,
        grid=(ne // BLK_E,),
        in_specs=[_rspec(BLK_E, HID), _wspec((HID, HID)), _wspec((1, HID)),
                  _wspec((HID, 16)), _wspec((1, 16))],
        out_specs=_rspec(BLK_E, 16),
        out_shape=_f32((ne, 16)),
    )(zb, w2, b2, w3, b3)


_MESH = plsc.VectorSubcoreMesh(core_axis_name="c", subcore_axis_name="s")


def _gather_pass(h, pos16, src, dst):
    ne = src.shape[0]
    ew = ne // (NC * NS)
    nch = ew // CHK
    src2 = src.reshape(-1, CHK)
    dst2 = dst.reshape(-1, CHK)

    @functools.partial(
        pl.kernel,
        out_type=[jax.ShapeDtypeStruct((ne, HID), jnp.bfloat16),
                  jax.ShapeDtypeStruct((ne, HID), jnp.bfloat16),
                  _f32((ne, 16))],
        mesh=_MESH,
        compiler_params=pltpu.CompilerParams(use_tc_tiling_on_sc=False),
        scratch_types=[
            pltpu.VMEM((nch, CHK), jnp.int32), pltpu.VMEM((nch, CHK), jnp.int32),
            pltpu.VMEM((2, CHK, HID), jnp.bfloat16),
            pltpu.VMEM((2, CHK, HID), jnp.bfloat16),
            pltpu.VMEM((CHK, 16), jnp.float32),
            pltpu.VMEM((CHK, 16), jnp.float32),
            pltpu.SemaphoreType.DMA, pltpu.SemaphoreType.DMA,
            pltpu.SemaphoreType.DMA,
        ],
    )
    def k(src_h, dst_h, h_h, pos_h, hs_o, hd_o, rel_o,
          ixs, ixd, hsv, hdv, psv, pdv, sA, sB, sP):
        c = lax.axis_index("c")
        s = lax.axis_index("s")
        wid = s * NC + c
        pltpu.sync_copy(src_h.at[pl.ds(wid * nch, nch)], ixs)
        pltpu.sync_copy(dst_h.at[pl.ds(wid * nch, nch)], ixd)

        def load(j, b, sem):
            pltpu.async_copy(h_h.at[ixs.at[j]], hsv.at[b], sem)
            pltpu.async_copy(h_h.at[ixd.at[j]], hdv.at[b], sem)

        def phase(j, b, sem):
            pltpu.make_async_copy(h_h.at[pl.ds(0, CHK)], hsv.at[b], sem).wait()
            pltpu.make_async_copy(h_h.at[pl.ds(0, CHK)], hdv.at[b], sem).wait()

            @pl.when(j + 1 < nch)
            def _():
                load(j + 1, 1 - b, sB if b == 0 else sA)
            cp3 = pltpu.async_copy(pos_h.at[ixs.at[j]], psv, sP)
            cp4 = pltpu.async_copy(pos_h.at[ixd.at[j]], pdv, sP)
            cp3.wait()
            cp4.wait()

            def sub(r, _2):
                psv[r] = psv[r] - pdv[r]
                return 0
            lax.fori_loop(0, CHK, sub, 0)
            base = wid * ew + j * CHK
            pltpu.sync_copy(hsv.at[b], hs_o.at[pl.ds(base, CHK)])
            pltpu.sync_copy(hdv.at[b], hd_o.at[pl.ds(base, CHK)])
            pltpu.sync_copy(psv, rel_o.at[pl.ds(base, CHK)])

        load(0, 0, sA)

        def pair(b2, _):
            phase(b2 * 2, 0, sA)
            phase(b2 * 2 + 1, 1, sB)
            return 0
        lax.fori_loop(0, nch // 2, pair, 0)
        if nch % 2 == 1:
            phase(nch - 1, 0, sA)

    return k(src2, dst2, h, pos16)


def _scatter_pass(m, pr16, dst, n):
    ne = m.shape[0]
    ew = ne // (NC * NS)
    nch = ew // CHKS
    rpt = n // NS
    zr = 25
    dst2 = dst.reshape(-1, CHKS)

    @functools.partial(
        pl.kernel,
        out_type=[_f32((NC, n, HID)), _f32((NC, n, 16))],
        mesh=_MESH,
        compiler_params=pltpu.CompilerParams(use_tc_tiling_on_sc=False),
        scratch_types=[
            pltpu.VMEM((nch, CHKS), jnp.int32),
            pltpu.VMEM((2, CHKS, HID), jnp.float32),
            pltpu.VMEM((2, CHKS, 16), jnp.float32),
            pltpu.VMEM((zr, HID), jnp.float32),
            pltpu.VMEM((zr, 16), jnp.float32),
            pltpu.VMEM_SHARED((n, HID), jnp.float32),
            pltpu.VMEM_SHARED((n, 16), jnp.float32),
            pltpu.SemaphoreType.DMA, pltpu.SemaphoreType.DMA,
        ],
    )
    def k(dst_h, m_h, pr_h, agg_o, pu_o, ixd, mv, prv, zba, zbp, aggs, pus,
          sA, sB):
        c = lax.axis_index("c")
        s = lax.axis_index("s")
        wid = s * NC + c

        def zrow(r, _):
            for g in range(HID // 16):
                zba[r, pl.ds(g * 16, 16)] = jnp.zeros((16,), jnp.float32)
            zbp[r] = jnp.zeros((16,), jnp.float32)
            return 0
        lax.fori_loop(0, zr, zrow, 0)
        pltpu.sync_copy(dst_h.at[pl.ds(wid * nch, nch)], ixd)

        def zcp(b, _):
            r0 = s * rpt + b * zr
            pltpu.sync_copy(zba, aggs.at[pl.ds(r0, zr)])
            pltpu.sync_copy(zbp, pus.at[pl.ds(r0, zr)])
            return 0
        lax.fori_loop(0, rpt // zr, zcp, 0)
        plsc.subcore_barrier()

        def load(j, b, sem):
            base = wid * ew + j * CHKS
            pltpu.async_copy(m_h.at[pl.ds(base, CHKS)], mv.at[b], sem)
            pltpu.async_copy(pr_h.at[pl.ds(base, CHKS)], prv.at[b], sem)

        def drain(j, b, sem):
            pltpu.make_async_copy(m_h.at[pl.ds(0, CHKS)], mv.at[b], sem).wait()
            pltpu.make_async_copy(pr_h.at[pl.ds(0, CHKS)], prv.at[b], sem).wait()
            pltpu.sync_copy(mv.at[b], aggs.at[ixd.at[j]], add=True)
            pltpu.sync_copy(prv.at[b], pus.at[ixd.at[j]], add=True)

        load(0, 0, sA)

        def pair(b2, _):
            j = b2 * 2
            load(j + 1, 1, sB)
            drain(j, 0, sA)

            @pl.when(j + 2 < nch)
            def _():
                load(j + 2, 0, sA)
            drain(j + 1, 1, sB)
            return 0
        lax.fori_loop(0, nch // 2, pair, 0)
        if nch % 2 == 1:
            drain(nch - 1, 0, sA)
        plsc.subcore_barrier()

        def rcp(b, _):
            r0 = s * rpt + b * zr
            pltpu.sync_copy(aggs.at[pl.ds(r0, zr)], zba)
            pltpu.sync_copy(zba, agg_o.at[c, pl.ds(r0, zr)])
            pltpu.sync_copy(pus.at[pl.ds(r0, zr)], zbp)
            pltpu.sync_copy(zbp, pu_o.at[c, pl.ds(r0, zr)])
            return 0
        lax.fori_loop(0, rpt // zr, rcp, 0)

    return k(dst2, m, pr16)


def _bond_gather(s_table, t_table, src, dst):
    ne = src.shape[0]
    ew = ne // (NC * NS)
    nch = ew // CHK
    src2 = src.reshape(-1, CHK)
    dst2 = dst.reshape(-1, CHK)

    @functools.partial(
        pl.kernel,
        out_type=_f32((ne, HID)),
        mesh=_MESH,
        compiler_params=pltpu.CompilerParams(use_tc_tiling_on_sc=False),
        scratch_types=[
            pltpu.VMEM((nch, CHK), jnp.int32), pltpu.VMEM((nch, CHK), jnp.int32),
            pltpu.VMEM((2, CHK, HID), jnp.float32),
            pltpu.VMEM((2, CHK, HID), jnp.float32),
            pltpu.SemaphoreType.DMA, pltpu.SemaphoreType.DMA,
        ],
    )
    def k(src_h, dst_h, st_h, tt_h, zb_o, ixs, ixd, sv, tv, sA, sB):
        c = lax.axis_index("c")
        s = lax.axis_index("s")
        wid = s * NC + c
        pltpu.sync_copy(src_h.at[pl.ds(wid * nch, nch)], ixs)
        pltpu.sync_copy(dst_h.at[pl.ds(wid * nch, nch)], ixd)

        def load(j, b, sem):
            pltpu.async_copy(st_h.at[ixs.at[j]], sv.at[b], sem)
            pltpu.async_copy(tt_h.at[ixd.at[j]], tv.at[b], sem)

        def phase(j, b, sem):
            pltpu.make_async_copy(st_h.at[pl.ds(0, CHK)], sv.at[b], sem).wait()
            pltpu.make_async_copy(st_h.at[pl.ds(0, CHK)], tv.at[b], sem).wait()

            @pl.when(j + 1 < nch)
            def _():
                load(j + 1, 1 - b, sB if b == 0 else sA)

            def addrow(r, _2):
                for g in range(HID // 16):
                    sl = pl.ds(g * 16, 16)
                    sv[b, r, sl] = sv[b, r, sl] + tv[b, r, sl]
                return 0
            lax.fori_loop(0, CHK, addrow, 0)
            base = wid * ew + j * CHK
            pltpu.sync_copy(sv.at[b], zb_o.at[pl.ds(base, CHK)])

        load(0, 0, sA)

        def pair(b2, _):
            phase(b2 * 2, 0, sA)
            phase(b2 * 2 + 1, 1, sB)
            return 0
        lax.fori_loop(0, nch // 2, pair, 0)
        if nch % 2 == 1:
            phase(nch - 1, 0, sA)

    return k(src2, dst2, s_table, t_table)


def kernel(x, edge_index, edge_attr, pos, batch, t, params):
    n = x.shape[0]
    src = edge_index[0]
    dst = edge_index[1]

    half = TIME_DIM // 2
    freqs = jnp.exp(-np.log(10000.0) * jnp.arange(half, dtype=jnp.float32) / half)
    targs = t[:, None].astype(jnp.float32) * freqs[None, :]
    time_emb = jnp.concatenate([jnp.sin(targs), jnp.cos(targs)], axis=-1)
    teN = time_emb[batch]

    pos4 = jnp.pad(pos, ((0, 0), (0, 13)))

    def row(b):
        return b.reshape(1, -1)

    h, hbf = _node_enc(x, params["node_enc"]["W"], row(params["node_enc"]["b"]))
    e = _edge_enc(edge_attr, params["edge_enc"]["W"], row(params["edge_enc"]["b"]))

    for lp in params["layers"]:
        w1 = lp["edge1"]["W"]
        w1de = jnp.pad(w1[2 * HID:], ((0, HID - 1), (0, 0)))
        hs, hd, rel16 = _gather_pass(hbf, pos4, src, dst)
        m, e, pr16 = _edge_pass(
            hs, hd, e, rel16,
            w1[0:2 * HID], w1de, row(lp["edge1"]["b"]),
            lp["edge2"]["W"], row(lp["edge2"]["b"]),
            lp["coord"]["W"], lp["coord"]["b"].reshape(1, 1))
        agg2, pu2 = _scatter_pass(m, pr16, dst, n)
        h, pos4, hbf = _node_pass(
            h, agg2, pos4, pu2,
            lp["node1"]["W"], row(lp["node1"]["b"]),
            lp["node2"]["W"], row(lp["node2"]["b"]))

    ap, pp, bp = params["atom_pred"], params["pos_pred"], params["bond_pred"]
    wp3 = jnp.pad(pp[2]["W"], ((0, 0), (0, 8 - 3)))
    bp3 = jnp.pad(pp[2]["b"], (0, 8 - 3)).reshape(1, 8)
    wb1 = bp[0]["W"]
    atom_logits, posn8, s_table, t_table = _final_node(
        h, teN,
        ap[0]["W"], row(ap[0]["b"]), ap[1]["W"], row(ap[1]["b"]),
        ap[2]["W"], row(ap[2]["b"]),
        pp[0]["W"], row(pp[0]["b"]), pp[1]["W"], row(pp[1]["b"]), wp3, bp3,
        wb1[0:HID], wb1[2 * HID:], row(bp[0]["b"]), wb1[HID:2 * HID])

    zb = _bond_gather(s_table, t_table, src, dst)
    bond_logits = _bond_pass(zb, bp[1]["W"], row(bp[1]["b"]),
                             bp[2]["W"], row(bp[2]["b"]))

    return atom_logits, posn8[:, :3], bond_logits


# speedup vs baseline: 1.3198x; 1.3198x over previous
import functools

import jax
import jax.numpy as jnp
import numpy as np
from jax import lax
from jax.experimental import pallas as pl
from jax.experimental.pallas import tpu as pltpu
from jax.experimental.pallas import tpu_sc as plsc

HID = 128
TIME_DIM = 128

BLK_E = 512
BLK_N = 1000

NC, NS = 2, 16
CHK = 200
CHKS = 80


def _silu(x):
    return x * jax.nn.sigmoid(x)


def _dot(a, b):
    return jnp.dot(a, b, preferred_element_type=jnp.float32)


def _wspec(shape):
    return pl.BlockSpec(shape, lambda i: (0,) * len(shape))


def _rspec(blk, ncols):
    return pl.BlockSpec((blk, ncols), lambda i: (i, 0))


def _node_enc_body(x_ref, wne, bne, h_ref):
    h_ref[...] = _dot(x_ref[...], wne[...]) + bne[...]


def _edge_enc_body(ea_ref, wee, bee, e_ref):
    e_ref[...] = _dot(ea_ref[...], wee[...]) + bee[...]


def _edge_body(hs_ref, hd_ref, e_ref, rel_ref, w1hh, w1de, b1, w2, b2, wc, bc,
               m_ref, eo_ref, pr_ref):
    e = e_ref[...]
    rel = rel_ref[...]
    d2 = jnp.sum(rel * rel, axis=1, keepdims=True)
    hh = jnp.concatenate([hs_ref[...], hd_ref[...]], axis=1)
    de = jnp.concatenate(
        [d2, e, jnp.zeros((e.shape[0], HID - 1), jnp.float32)], axis=1)
    t1 = _dot(hh, w1hh[...]) + _dot(de, w1de[...]) + b1[...]
    u = _silu(t1)
    m = _silu(_dot(u, w2[...]) + b2[...])
    cw = _dot(m, wc[...]) + bc[...]
    m_ref[...] = m
    eo_ref[...] = e + m
    one4 = (lax.broadcasted_iota(jnp.int32, (1, 16), 1) == 4).astype(jnp.float32)
    pr_ref[...] = rel * cw + one4


def _node_body(h_ref, a0_ref, a1_ref, pos_ref, p0_ref, p1_ref,
               wn1, bn1, wn2, bn2, h_o, pos_o):
    h = h_ref[...]
    agg = a0_ref[...] + a1_ref[...]
    pu = p0_ref[...] + p1_ref[...]
    hcat = jnp.concatenate([h, agg], axis=1)
    u = _silu(_dot(hcat, wn1[...]) + bn1[...])
    hn = h + _dot(u, wn2[...]) + bn2[...]
    h_o[...] = hn
    cnt = jnp.maximum(pu[:, 4:5], 1.0)
    mask3 = (lax.broadcasted_iota(jnp.int32, (1, 16), 1) < 3).astype(jnp.float32)
    pos_o[...] = pos_ref[...] + pu * mask3 / cnt


def _final_node_body(h_ref, te_ref,
                     wat1, bat1, wat2, bat2, wat3, bat3,
                     wp1, bp1, wp2, bp2, wp3, bp3,
                     wbs, wbte, bb1, wbt,
                     atom_o, pos_o, s_o, t_o):
    h = h_ref[...]
    te = te_ref[...]
    hcat = jnp.concatenate([h, te], axis=1)
    a1 = _silu(_dot(hcat, wat1[...]) + bat1[...])
    a2 = _silu(_dot(a1, wat2[...]) + bat2[...])
    atom_o[...] = _dot(a2, wat3[...]) + bat3[...]
    p1 = _silu(_dot(hcat, wp1[...]) + bp1[...])
    p2 = _silu(_dot(p1, wp2[...]) + bp2[...])
    pos_o[...] = _dot(p2, wp3[...]) + bp3[...]
    s_o[...] = _dot(h, wbs[...]) + _dot(te, wbte[...]) + bb1[...]
    t_o[...] = _dot(h, wbt[...])


def _bond_body(zb_ref, w2, b2, w3, b3, o_ref):
    u = _silu(zb_ref[...])
    u2 = _silu(_dot(u, w2[...]) + b2[...])
    o_ref[...] = _dot(u2, w3[...]) + b3[...]


def _f32(shape):
    return jax.ShapeDtypeStruct(shape, jnp.float32)


def _node_enc(x, wne, bne):
    n = x.shape[0]
    return pl.pallas_call(
        _node_enc_body,
        grid=(n // BLK_N,),
        in_specs=[_rspec(BLK_N, HID), _wspec((HID, HID)), _wspec((1, HID))],
        out_specs=_rspec(BLK_N, HID),
        out_shape=_f32((n, HID)),
    )(x, wne, bne)


def _edge_enc(ea, wee, bee):
    ne = ea.shape[0]
    return pl.pallas_call(
        _edge_enc_body,
        grid=(ne // BLK_E,),
        in_specs=[_rspec(BLK_E, 16), _wspec((16, HID)), _wspec((1, HID))],
        out_specs=_rspec(BLK_E, HID),
        out_shape=_f32((ne, HID)),
    )(ea, wee, bee)


def _edge_pass(hs, hd, e, rel16, w1hh, w1de, b1, w2, b2, wc, bc):
    ne = e.shape[0]
    return pl.pallas_call(
        _edge_body,
        grid=(ne // BLK_E,),
        in_specs=[_rspec(BLK_E, HID), _rspec(BLK_E, HID), _rspec(BLK_E, HID),
                  _rspec(BLK_E, 16),
                  _wspec((2 * HID, HID)), _wspec((2 * HID, HID)), _wspec((1, HID)),
                  _wspec((HID, HID)), _wspec((1, HID)),
                  _wspec((HID, 1)), _wspec((1, 1))],
        out_specs=[_rspec(BLK_E, HID), _rspec(BLK_E, HID), _rspec(BLK_E, 16)],
        out_shape=[_f32((ne, HID)), _f32((ne, HID)), _f32((ne, 16))],
    )(hs, hd, e, rel16, w1hh, w1de, b1, w2, b2, wc, bc)


def _node_pass(h, agg2, pos4, pu2, wn1, bn1, wn2, bn2):
    n = h.shape[0]
    return pl.pallas_call(
        _node_body,
        grid=(n // BLK_N,),
        in_specs=[_rspec(BLK_N, HID), _rspec(BLK_N, HID), _rspec(BLK_N, HID),
                  _rspec(BLK_N, 16), _rspec(BLK_N, 16), _rspec(BLK_N, 16),
                  _wspec((2 * HID, HID)), _wspec((1, HID)),
                  _wspec((HID, HID)), _wspec((1, HID))],
        out_specs=[_rspec(BLK_N, HID), _rspec(BLK_N, 16)],
        out_shape=[_f32((n, HID)), _f32((n, 16))],
    )(h, agg2[0], agg2[1], pos4, pu2[0], pu2[1], wn1, bn1, wn2, bn2)


def _final_node(h, te, *ws):
    n = h.shape[0]
    wspecs = [_wspec(w.shape) for w in ws]
    return pl.pallas_call(
        _final_node_body,
        grid=(n // BLK_N,),
        in_specs=[_rspec(BLK_N, HID), _rspec(BLK_N, HID)] + wspecs,
        out_specs=[_rspec(BLK_N, HID), _rspec(BLK_N, 8),
                   _rspec(BLK_N, HID), _rspec(BLK_N, HID)],
        out_shape=[_f32((n, HID)), _f32((n, 8)), _f32((n, HID)), _f32((n, HID))],
    )(h, te, *ws)


def _bond_pass(zb, w2, b2, w3, b3):
    ne = zb.shape[0]
    return pl.pallas_call(
        _bond_body,
        grid=(ne // BLK_E,),
        in_specs=[_rspec(BLK_E, HID), _wspec((HID, HID)), _wspec((1, HID)),
                  _wspec((HID, 16)), _wspec((1, 16))],
        out_specs=_rspec(BLK_E, 16),
        out_shape=_f32((ne, 16)),
    )(zb, w2, b2, w3, b3)


_MESH = plsc.VectorSubcoreMesh(core_axis_name="c", subcore_axis_name="s")


def _gather_pass(h, pos16, src, dst):
    ne = src.shape[0]
    ew = ne // (NC * NS)
    nch = ew // CHK
    src2 = src.reshape(-1, CHK)
    dst2 = dst.reshape(-1, CHK)

    @functools.partial(
        pl.kernel,
        out_type=[_f32((ne, HID)), _f32((ne, HID)), _f32((ne, 16))],
        mesh=_MESH,
        compiler_params=pltpu.CompilerParams(use_tc_tiling_on_sc=False),
        scratch_types=[
            pltpu.VMEM((nch, CHK), jnp.int32), pltpu.VMEM((nch, CHK), jnp.int32),
            pltpu.VMEM((2, CHK, HID), jnp.float32),
            pltpu.VMEM((2, CHK, HID), jnp.float32),
            pltpu.VMEM((CHK, 16), jnp.float32),
            pltpu.VMEM((CHK, 16), jnp.float32),
            pltpu.SemaphoreType.DMA, pltpu.SemaphoreType.DMA,
            pltpu.SemaphoreType.DMA,
            pltpu.SemaphoreType.DMA, pltpu.SemaphoreType.DMA,
        ],
    )
    def k(src_h, dst_h, h_h, pos_h, hs_o, hd_o, rel_o,
          ixs, ixd, hsv, hdv, psv, pdv, sA, sB, sP, sWA, sWB):
        c = lax.axis_index("c")
        s = lax.axis_index("s")
        wid = s * NC + c
        pltpu.sync_copy(src_h.at[pl.ds(wid * nch, nch)], ixs)
        pltpu.sync_copy(dst_h.at[pl.ds(wid * nch, nch)], ixd)

        def wdrain(b, wsem):
            pltpu.make_async_copy(h_h.at[pl.ds(0, CHK)], hsv.at[b], wsem).wait()
            pltpu.make_async_copy(h_h.at[pl.ds(0, CHK)], hdv.at[b], wsem).wait()

        def load(j, b, sem, wsem):
            @pl.when(j >= 2)
            def _():
                wdrain(b, wsem)
            pltpu.async_copy(h_h.at[ixs.at[j]], hsv.at[b], sem)
            pltpu.async_copy(h_h.at[ixd.at[j]], hdv.at[b], sem)

        def phase(j, b, sem, wsem):
            pltpu.make_async_copy(h_h.at[pl.ds(0, CHK)], hsv.at[b], sem).wait()
            pltpu.make_async_copy(h_h.at[pl.ds(0, CHK)], hdv.at[b], sem).wait()

            @pl.when(j + 1 < nch)
            def _():
                load(j + 1, 1 - b, sB if b == 0 else sA,
                     sWB if b == 0 else sWA)
            cp3 = pltpu.async_copy(pos_h.at[ixs.at[j]], psv, sP)
            cp4 = pltpu.async_copy(pos_h.at[ixd.at[j]], pdv, sP)
            cp3.wait()
            cp4.wait()

            def sub(r, _2):
                psv[r] = psv[r] - pdv[r]
                return 0
            lax.fori_loop(0, CHK, sub, 0)
            base = wid * ew + j * CHK
            pltpu.async_copy(hsv.at[b], hs_o.at[pl.ds(base, CHK)], wsem)
            pltpu.async_copy(hdv.at[b], hd_o.at[pl.ds(base, CHK)], wsem)
            pltpu.sync_copy(psv, rel_o.at[pl.ds(base, CHK)])

        load(0, 0, sA, sWA)

        def pair(b2, _):
            phase(b2 * 2, 0, sA, sWA)
            phase(b2 * 2 + 1, 1, sB, sWB)
            return 0
        lax.fori_loop(0, nch // 2, pair, 0)
        if nch % 2 == 1:
            phase(nch - 1, 0, sA, sWA)
        wdrain(0, sWA)
        wdrain(1, sWB)

    return k(src2, dst2, h, pos16)


def _scatter_pass(m, pr16, dst, n):
    ne = m.shape[0]
    ew = ne // (NC * NS)
    nch = ew // CHKS
    rpt = n // NS
    zr = 25
    dst2 = dst.reshape(-1, CHKS)

    @functools.partial(
        pl.kernel,
        out_type=[_f32((NC, n, HID)), _f32((NC, n, 16))],
        mesh=_MESH,
        compiler_params=pltpu.CompilerParams(use_tc_tiling_on_sc=False),
        scratch_types=[
            pltpu.VMEM((nch, CHKS), jnp.int32),
            pltpu.VMEM((2, CHKS, HID), jnp.float32),
            pltpu.VMEM((2, CHKS, 16), jnp.float32),
            pltpu.VMEM((zr, HID), jnp.float32),
            pltpu.VMEM((zr, 16), jnp.float32),
            pltpu.VMEM_SHARED((n, HID), jnp.float32),
            pltpu.VMEM_SHARED((n, 16), jnp.float32),
            pltpu.SemaphoreType.DMA, pltpu.SemaphoreType.DMA,
        ],
    )
    def k(dst_h, m_h, pr_h, agg_o, pu_o, ixd, mv, prv, zba, zbp, aggs, pus,
          sA, sB):
        c = lax.axis_index("c")
        s = lax.axis_index("s")
        wid = s * NC + c

        def zrow(r, _):
            for g in range(HID // 16):
                zba[r, pl.ds(g * 16, 16)] = jnp.zeros((16,), jnp.float32)
            zbp[r] = jnp.zeros((16,), jnp.float32)
            return 0
        lax.fori_loop(0, zr, zrow, 0)
        pltpu.sync_copy(dst_h.at[pl.ds(wid * nch, nch)], ixd)

        def zcp(b, _):
            r0 = s * rpt + b * zr
            pltpu.sync_copy(zba, aggs.at[pl.ds(r0, zr)])
            pltpu.sync_copy(zbp, pus.at[pl.ds(r0, zr)])
            return 0
        lax.fori_loop(0, rpt // zr, zcp, 0)
        plsc.subcore_barrier()

        def load(j, b, sem):
            base = wid * ew + j * CHKS
            pltpu.async_copy(m_h.at[pl.ds(base, CHKS)], mv.at[b], sem)
            pltpu.async_copy(pr_h.at[pl.ds(base, CHKS)], prv.at[b], sem)

        def drain(j, b, sem):
            pltpu.make_async_copy(m_h.at[pl.ds(0, CHKS)], mv.at[b], sem).wait()
            pltpu.make_async_copy(pr_h.at[pl.ds(0, CHKS)], prv.at[b], sem).wait()
            pltpu.sync_copy(mv.at[b], aggs.at[ixd.at[j]], add=True)
            pltpu.sync_copy(prv.at[b], pus.at[ixd.at[j]], add=True)

        load(0, 0, sA)

        def pair(b2, _):
            j = b2 * 2
            load(j + 1, 1, sB)
            drain(j, 0, sA)

            @pl.when(j + 2 < nch)
            def _():
                load(j + 2, 0, sA)
            drain(j + 1, 1, sB)
            return 0
        lax.fori_loop(0, nch // 2, pair, 0)
        if nch % 2 == 1:
            drain(nch - 1, 0, sA)
        plsc.subcore_barrier()

        def rcp(b, _):
            r0 = s * rpt + b * zr
            pltpu.sync_copy(aggs.at[pl.ds(r0, zr)], zba)
            pltpu.sync_copy(zba, agg_o.at[c, pl.ds(r0, zr)])
            pltpu.sync_copy(pus.at[pl.ds(r0, zr)], zbp)
            pltpu.sync_copy(zbp, pu_o.at[c, pl.ds(r0, zr)])
            return 0
        lax.fori_loop(0, rpt // zr, rcp, 0)

    return k(dst2, m, pr16)


def _bond_gather(s_table, t_table, src, dst):
    ne = src.shape[0]
    ew = ne // (NC * NS)
    nch = ew // CHK
    src2 = src.reshape(-1, CHK)
    dst2 = dst.reshape(-1, CHK)

    @functools.partial(
        pl.kernel,
        out_type=_f32((ne, HID)),
        mesh=_MESH,
        compiler_params=pltpu.CompilerParams(use_tc_tiling_on_sc=False),
        scratch_types=[
            pltpu.VMEM((nch, CHK), jnp.int32), pltpu.VMEM((nch, CHK), jnp.int32),
            pltpu.VMEM((2, CHK, HID), jnp.float32),
            pltpu.VMEM((2, CHK, HID), jnp.float32),
            pltpu.SemaphoreType.DMA, pltpu.SemaphoreType.DMA,
            pltpu.SemaphoreType.DMA, pltpu.SemaphoreType.DMA,
        ],
    )
    def k(src_h, dst_h, st_h, tt_h, zb_o, ixs, ixd, sv, tv, sA, sB, sWA, sWB):
        c = lax.axis_index("c")
        s = lax.axis_index("s")
        wid = s * NC + c
        pltpu.sync_copy(src_h.at[pl.ds(wid * nch, nch)], ixs)
        pltpu.sync_copy(dst_h.at[pl.ds(wid * nch, nch)], ixd)

        def wdrain(b, wsem):
            pltpu.make_async_copy(st_h.at[pl.ds(0, CHK)], sv.at[b], wsem).wait()

        def load(j, b, sem, wsem):
            @pl.when(j >= 2)
            def _():
                wdrain(b, wsem)
            pltpu.async_copy(st_h.at[ixs.at[j]], sv.at[b], sem)
            pltpu.async_copy(tt_h.at[ixd.at[j]], tv.at[b], sem)

        def phase(j, b, sem, wsem):
            pltpu.make_async_copy(st_h.at[pl.ds(0, CHK)], sv.at[b], sem).wait()
            pltpu.make_async_copy(st_h.at[pl.ds(0, CHK)], tv.at[b], sem).wait()

            @pl.when(j + 1 < nch)
            def _():
                load(j + 1, 1 - b, sB if b == 0 else sA,
                     sWB if b == 0 else sWA)

            def addrow(r, _2):
                for g in range(HID // 16):
                    sl = pl.ds(g * 16, 16)
                    sv[b, r, sl] = sv[b, r, sl] + tv[b, r, sl]
                return 0
            lax.fori_loop(0, CHK, addrow, 0)
            base = wid * ew + j * CHK
            pltpu.async_copy(sv.at[b], zb_o.at[pl.ds(base, CHK)], wsem)

        load(0, 0, sA, sWA)

        def pair(b2, _):
            phase(b2 * 2, 0, sA, sWA)
            phase(b2 * 2 + 1, 1, sB, sWB)
            return 0
        lax.fori_loop(0, nch // 2, pair, 0)
        if nch % 2 == 1:
            phase(nch - 1, 0, sA, sWA)
        wdrain(0, sWA)
        wdrain(1, sWB)

    return k(src2, dst2, s_table, t_table)


def kernel(x, edge_index, edge_attr, pos, batch, t, params):
    n = x.shape[0]
    src = edge_index[0]
    dst = edge_index[1]

    half = TIME_DIM // 2
    freqs = jnp.exp(-np.log(10000.0) * jnp.arange(half, dtype=jnp.float32) / half)
    targs = t[:, None].astype(jnp.float32) * freqs[None, :]
    time_emb = jnp.concatenate([jnp.sin(targs), jnp.cos(targs)], axis=-1)
    teN = time_emb[batch]

    pos4 = jnp.pad(pos, ((0, 0), (0, 13)))

    def row(b):
        return b.reshape(1, -1)

    h = _node_enc(x, params["node_enc"]["W"], row(params["node_enc"]["b"]))
    e = _edge_enc(edge_attr, params["edge_enc"]["W"], row(params["edge_enc"]["b"]))

    for lp in params["layers"]:
        w1 = lp["edge1"]["W"]
        w1de = jnp.pad(w1[2 * HID:], ((0, HID - 1), (0, 0)))
        hs, hd, rel16 = _gather_pass(h, pos4, src, dst)
        m, e, pr16 = _edge_pass(
            hs, hd, e, rel16,
            w1[0:2 * HID], w1de, row(lp["edge1"]["b"]),
            lp["edge2"]["W"], row(lp["edge2"]["b"]),
            lp["coord"]["W"], lp["coord"]["b"].reshape(1, 1))
        agg2, pu2 = _scatter_pass(m, pr16, dst, n)
        h, pos4 = _node_pass(
            h, agg2, pos4, pu2,
            lp["node1"]["W"], row(lp["node1"]["b"]),
            lp["node2"]["W"], row(lp["node2"]["b"]))

    ap, pp, bp = params["atom_pred"], params["pos_pred"], params["bond_pred"]
    wp3 = jnp.pad(pp[2]["W"], ((0, 0), (0, 8 - 3)))
    bp3 = jnp.pad(pp[2]["b"], (0, 8 - 3)).reshape(1, 8)
    wb1 = bp[0]["W"]
    atom_logits, posn8, s_table, t_table = _final_node(
        h, teN,
        ap[0]["W"], row(ap[0]["b"]), ap[1]["W"], row(ap[1]["b"]),
        ap[2]["W"], row(ap[2]["b"]),
        pp[0]["W"], row(pp[0]["b"]), pp[1]["W"], row(pp[1]["b"]), wp3, bp3,
        wb1[0:HID], wb1[2 * HID:], row(bp[0]["b"]), wb1[HID:2 * HID])

    zb = _bond_gather(s_table, t_table, src, dst)
    bond_logits = _bond_pass(zb, bp[1]["W"], row(bp[1]["b"]),
                             bp[2]["W"], row(bp[2]["b"]))

    return atom_logits, posn8[:, :3], bond_logits
